# initial kernel scaffold (unmeasured)
import jax
import jax.numpy as jnp
from jax import lax
from jax.experimental import pallas as pl
from jax.experimental.pallas import tpu as pltpu

T = 2048
V_LOCAL = 16384
D = 1024


def kernel(ids, E):
    def body(ids_ref, e_ref, out_ref, local_sem, send_sem, recv_sem):
        my_x = lax.axis_index("x")
        my_y = lax.axis_index("y")
        my_z = lax.axis_index("z")
        partner = (1 - my_x, my_y, my_z)

        barrier = pltpu.get_barrier_semaphore()
        pl.semaphore_signal(
            barrier, inc=1, device_id=partner,
            device_id_type=pl.DeviceIdType.MESH,
        )
        pl.semaphore_wait(barrier, 1)

        def issue(i, n_mine):
            tok = ids_ref[i]
            owner = tok // V_LOCAL
            row = tok - owner * V_LOCAL
            owned = owner == my_x

            @pl.when(owned)
            def _():
                pltpu.make_async_copy(
                    e_ref.at[pl.ds(row, 1), :],
                    out_ref.at[pl.ds(i, 1), :],
                    local_sem,
                ).start()
                pltpu.make_async_remote_copy(
                    src_ref=e_ref.at[pl.ds(row, 1), :],
                    dst_ref=out_ref.at[pl.ds(i, 1), :],
                    send_sem=send_sem,
                    recv_sem=recv_sem,
                    device_id=partner,
                    device_id_type=pl.DeviceIdType.MESH,
                ).start()

            return n_mine + jnp.where(owned, 1, 0)

        n_mine = lax.fori_loop(0, T, issue, jnp.int32(0))
        n_theirs = T - n_mine

        pl.semaphore_wait(local_sem, n_mine)
        pl.semaphore_wait(send_sem, n_mine)
        pl.semaphore_wait(recv_sem, n_theirs)

    return pl.pallas_call(
        body,
        out_shape=jax.ShapeDtypeStruct((T, D), jnp.float32),
        in_specs=[
            pl.BlockSpec(memory_space=pltpu.SMEM),
            pl.BlockSpec(memory_space=pltpu.ANY),
        ],
        out_specs=pl.BlockSpec(memory_space=pltpu.VMEM),
        scratch_shapes=[
            pltpu.SemaphoreType.DMA,
            pltpu.SemaphoreType.DMA,
            pltpu.SemaphoreType.DMA,
        ],
        compiler_params=pltpu.CompilerParams(collective_id=0),
    )(ids, E)


# baseline (device time: 99430 ns/iter reference)
import jax
import jax.numpy as jnp
from jax import lax
from jax.experimental import pallas as pl
from jax.experimental.pallas import tpu as pltpu

T = 2048
V_LOCAL = 16384
D = 1024


def kernel(ids, E):
    def body(ids_ref, e_ref, out_ref, local_sem, send_sem, recv_sem):
        my_x = lax.axis_index("x")
        my_y = lax.axis_index("y")
        my_z = lax.axis_index("z")
        partner = (1 - my_x, my_y, my_z)

        barrier = pltpu.get_barrier_semaphore()
        pl.semaphore_signal(
            barrier, inc=1, device_id=partner,
            device_id_type=pl.DeviceIdType.MESH,
        )
        pl.semaphore_wait(barrier, 1)

        def issue(i, n_mine):
            tok = ids_ref[i]
            owner = tok // V_LOCAL
            row = tok - owner * V_LOCAL
            owned = owner == my_x

            @pl.when(owned)
            def _():
                pltpu.make_async_copy(
                    e_ref.at[pl.ds(row, 1), :],
                    out_ref.at[pl.ds(i, 1), :],
                    local_sem,
                ).start()
                pltpu.make_async_remote_copy(
                    src_ref=e_ref.at[pl.ds(row, 1), :],
                    dst_ref=out_ref.at[pl.ds(i, 1), :],
                    send_sem=send_sem,
                    recv_sem=recv_sem,
                    device_id=partner,
                    device_id_type=pl.DeviceIdType.MESH,
                ).start()

            return n_mine + jnp.where(owned, 1, 0)

        n_mine = lax.fori_loop(0, T, issue, jnp.int32(0))
        n_theirs = T - n_mine

        def drain(i, _):
            local_row = pltpu.make_async_copy(
                e_ref.at[pl.ds(0, 1), :], out_ref.at[pl.ds(0, 1), :], local_sem
            )
            remote_row = pltpu.make_async_remote_copy(
                src_ref=e_ref.at[pl.ds(0, 1), :],
                dst_ref=out_ref.at[pl.ds(0, 1), :],
                send_sem=send_sem,
                recv_sem=recv_sem,
                device_id=partner,
                device_id_type=pl.DeviceIdType.MESH,
            )

            @pl.when(i < n_mine)
            def _():
                local_row.wait()
                remote_row.wait_send()

            @pl.when(i < n_theirs)
            def _():
                remote_row.wait_recv()

            return 0

        lax.fori_loop(0, T, drain, 0)

    return pl.pallas_call(
        body,
        out_shape=jax.ShapeDtypeStruct((T, D), jnp.float32),
        in_specs=[
            pl.BlockSpec(memory_space=pltpu.SMEM),
            pl.BlockSpec(memory_space=pl.ANY),
        ],
        out_specs=pl.BlockSpec(memory_space=pltpu.VMEM),
        scratch_shapes=[
            pltpu.SemaphoreType.DMA,
            pltpu.SemaphoreType.DMA,
            pltpu.SemaphoreType.DMA,
        ],
        compiler_params=pltpu.CompilerParams(collective_id=0),
    )(ids, E)


# device time: 72066 ns/iter; 1.3797x vs baseline; 1.3797x over previous
import jax
import jax.numpy as jnp
from jax import lax
from jax.experimental import pallas as pl
from jax.experimental.pallas import tpu as pltpu

T = 2048
V_LOCAL = 16384
D = 1024


def kernel(ids, E):
    def body(ids_ref, e_ref, out_ref, local_sem, send_sem, recv_sem):
        my_x = lax.axis_index("x")
        my_y = lax.axis_index("y")
        my_z = lax.axis_index("z")
        partner = (1 - my_x, my_y, my_z)

        barrier = pltpu.get_barrier_semaphore()
        pl.semaphore_signal(
            barrier, inc=1, device_id=partner,
            device_id_type=pl.DeviceIdType.MESH,
        )
        pl.semaphore_wait(barrier, 1)

        base = my_x * V_LOCAL
        UNROLL = 8

        def issue(j, n_mine):
            for u in range(UNROLL):
                i = j * UNROLL + u
                tok = ids_ref[i]
                row = tok - base
                owned = (row >= 0) & (row < V_LOCAL)

                @pl.when(owned)
                def _():
                    pltpu.make_async_copy(
                        e_ref.at[pl.ds(row, 1), :],
                        out_ref.at[pl.ds(i, 1), :],
                        local_sem,
                    ).start()
                    pltpu.make_async_remote_copy(
                        src_ref=e_ref.at[pl.ds(row, 1), :],
                        dst_ref=out_ref.at[pl.ds(i, 1), :],
                        send_sem=send_sem,
                        recv_sem=recv_sem,
                        device_id=partner,
                        device_id_type=pl.DeviceIdType.MESH,
                    ).start()

                n_mine = n_mine + jnp.where(owned, 1, 0)
            return n_mine

        n_mine = lax.fori_loop(0, T // UNROLL, issue, jnp.int32(0))
        n_theirs = T - n_mine

        def drain(sem, count, is_recv, is_remote):
            for k in reversed(range(T.bit_length())):
                w = 1 << k
                if w > T:
                    continue

                @pl.when((count & w) != 0)
                def _():
                    if is_remote:
                        d = pltpu.make_async_remote_copy(
                            src_ref=e_ref.at[pl.ds(0, w), :],
                            dst_ref=out_ref.at[pl.ds(0, w), :],
                            send_sem=send_sem,
                            recv_sem=recv_sem,
                            device_id=partner,
                            device_id_type=pl.DeviceIdType.MESH,
                        )
                        d.wait_recv() if is_recv else d.wait_send()
                    else:
                        pltpu.make_async_copy(
                            e_ref.at[pl.ds(0, w), :],
                            out_ref.at[pl.ds(0, w), :],
                            sem,
                        ).wait()

        drain(local_sem, n_mine, False, False)
        drain(send_sem, n_mine, False, True)
        drain(recv_sem, n_theirs, True, True)

    return pl.pallas_call(
        body,
        out_shape=jax.ShapeDtypeStruct((T, D), jnp.float32),
        in_specs=[
            pl.BlockSpec(memory_space=pltpu.SMEM),
            pl.BlockSpec(memory_space=pl.ANY),
        ],
        out_specs=pl.BlockSpec(memory_space=pltpu.VMEM),
        scratch_shapes=[
            pltpu.SemaphoreType.DMA,
            pltpu.SemaphoreType.DMA,
            pltpu.SemaphoreType.DMA,
        ],
        compiler_params=pltpu.CompilerParams(collective_id=0),
    )(ids, E)


# device time: 41964 ns/iter; 2.3694x vs baseline; 1.7173x over previous
import os

import jax
import jax.numpy as jnp
from jax import lax
from jax.experimental import pallas as pl
from jax.experimental.pallas import tpu as pltpu

T = 2048
V_LOCAL = 16384
D = 1024

_MODE = os.environ.get("KMODE", "full")


def kernel(ids, E):
    def body(ids_ref, e_ref, out_ref, local_sem, send_sem, recv_sem):
        my_x = lax.axis_index("x")
        my_y = lax.axis_index("y")
        my_z = lax.axis_index("z")
        partner = (1 - my_x, my_y, my_z)

        barrier = pltpu.get_barrier_semaphore()
        pl.semaphore_signal(
            barrier, inc=1, device_id=partner,
            device_id_type=pl.DeviceIdType.MESH,
        )
        pl.semaphore_wait(barrier, 1)

        base = my_x * V_LOCAL
        UNROLL = 8

        def issue(j, n_mine):
            for u in range(UNROLL):
                i = j * UNROLL + u
                tok = ids_ref[i]
                row = tok - base
                owned = (row >= 0) & (row < V_LOCAL)

                @pl.when(owned)
                def _():
                    if _MODE != "nolocal":
                        pltpu.make_async_copy(
                            e_ref.at[pl.ds(row, 1), :],
                            out_ref.at[pl.ds(i, 1), :],
                            local_sem,
                        ).start()
                    if _MODE != "nordma":
                        pltpu.make_async_remote_copy(
                            src_ref=e_ref.at[pl.ds(row, 1), :],
                            dst_ref=out_ref.at[pl.ds(i, 1), :],
                            send_sem=send_sem,
                            recv_sem=recv_sem,
                            device_id=partner,
                            device_id_type=pl.DeviceIdType.MESH,
                        ).start()

                n_mine = n_mine + jnp.where(owned, 1, 0)
            return n_mine

        n_mine = lax.fori_loop(0, T // UNROLL, issue, jnp.int32(0))
        n_theirs = T - n_mine

        def drain(sem, count, is_recv, is_remote):
            for k in reversed(range(T.bit_length())):
                w = 1 << k
                if w > T:
                    continue

                @pl.when((count & w) != 0)
                def _():
                    if is_remote:
                        d = pltpu.make_async_remote_copy(
                            src_ref=e_ref.at[pl.ds(0, w), :],
                            dst_ref=out_ref.at[pl.ds(0, w), :],
                            send_sem=send_sem,
                            recv_sem=recv_sem,
                            device_id=partner,
                            device_id_type=pl.DeviceIdType.MESH,
                        )
                        d.wait_recv() if is_recv else d.wait_send()
                    else:
                        pltpu.make_async_copy(
                            e_ref.at[pl.ds(0, w), :],
                            out_ref.at[pl.ds(0, w), :],
                            sem,
                        ).wait()

        if _MODE != "nolocal":
            drain(local_sem, n_mine, False, False)
        if _MODE != "nordma":
            drain(send_sem, n_mine, False, True)
            drain(recv_sem, n_theirs, True, True)

    return pl.pallas_call(
        body,
        out_shape=jax.ShapeDtypeStruct((T, D), jnp.float32),
        in_specs=[
            pl.BlockSpec(memory_space=pltpu.SMEM),
            pl.BlockSpec(memory_space=pl.ANY),
        ],
        out_specs=pl.BlockSpec(memory_space=pltpu.VMEM),
        scratch_shapes=[
            pltpu.SemaphoreType.DMA,
            pltpu.SemaphoreType.DMA,
            pltpu.SemaphoreType.DMA,
        ],
        compiler_params=pltpu.CompilerParams(collective_id=0),
    )(ids, E)
